# Initial kernel scaffold; baseline (speedup 1.0000x reference)
import functools

import jax
import jax.numpy as jnp
from jax import lax
from jax.experimental import pallas as pl
from jax.experimental.pallas import tpu as pltpu
from jax.experimental.pallas import tpu_sc as plsc

N_NODES = 10000
N_PAD = 10240
F_HALF = 128
R_BLK = 1000
NB = N_NODES // R_BLK
EPS = 1e-5

_MESH = dict(core_axis_name="c", subcore_axis_name="s")
NC, NS = 2, 16


def _deg_body(dst_hbm, ew_hbm, out_hbm, dst_v, ew_v, zbuf, acc, sem):
    c = lax.axis_index("c")
    s = lax.axis_index("s")
    pltpu.sync_copy(dst_hbm.at[c, s], dst_v)
    pltpu.sync_copy(ew_hbm.at[c, s], ew_v)

    def zb(i, _):
        zbuf[pl.ds(i * 16, 16)] = jnp.zeros((16,), jnp.float32)
        return 0
    lax.fori_loop(0, 40, zb, 0)
    pltpu.sync_copy(zbuf, acc.at[pl.ds(s * 640, 640)])
    plsc.subcore_barrier()

    nchunks = dst_v.shape[0]

    def body(j, _):
        pltpu.sync_copy(ew_v.at[j], acc.at[dst_v.at[j]], add=True)
        return 0
    lax.fori_loop(0, nchunks, body, 0)
    plsc.subcore_barrier()
    pltpu.sync_copy(acc.at[pl.ds(s * 640, 640)],
                    out_hbm.at[c, pl.ds(s * 640, 640)])


def _make_deg(nchunks):
    return functools.partial(
        pl.kernel,
        out_type=jax.ShapeDtypeStruct((NC, N_PAD), jnp.float32),
        mesh=plsc.VectorSubcoreMesh(**_MESH),
        scratch_types=[
            pltpu.VMEM((nchunks, 128), jnp.int32),
            pltpu.VMEM((nchunks, 128), jnp.float32),
            pltpu.VMEM((640,), jnp.float32),
            pltpu.VMEM_SHARED((N_PAD,), jnp.float32),
            pltpu.SemaphoreType.DMA,
        ],
    )(_deg_body)


def _agg_body(g_hbm, src_hbm, dst_hbm, ew_hbm, out_hbm,
              src_v, dst_v, ew_v, rows_v, zbuf, acc, sem):
    c = lax.axis_index("c")
    s = lax.axis_index("s")
    pltpu.sync_copy(src_hbm.at[c, s], src_v)
    pltpu.sync_copy(dst_hbm.at[s], dst_v)
    pltpu.sync_copy(ew_hbm.at[s], ew_v)

    def zb(r, _):
        for f in range(8):
            zbuf[r, pl.ds(f * 16, 16)] = jnp.zeros((16,), jnp.float32)
        return 0
    lax.fori_loop(0, 125, zb, 0)
    for k in range(5):
        pltpu.sync_copy(zbuf, acc.at[pl.ds(s * 625 + k * 125, 125)])
    plsc.subcore_barrier()

    nchunks = src_v.shape[0]

    def chunk(j, _):
        pltpu.async_copy(g_hbm.at[src_v.at[j]], rows_v, sem).wait()

        def edge(e, _2):
            w = ew_v[j, e]
            for f in range(8):
                sl = pl.ds(f * 16, 16)
                rows_v[e, sl] = rows_v[e, sl] * w
            return 0
        lax.fori_loop(0, 128, edge, 0)
        pltpu.sync_copy(rows_v, acc.at[dst_v.at[j]], add=True)
        return 0
    lax.fori_loop(0, nchunks, chunk, 0)
    plsc.subcore_barrier()

    def dr(k, _):
        pltpu.sync_copy(acc.at[pl.ds(s * 625 + k * 125, 125)],
                        out_hbm.at[pl.ds(c * N_NODES + s * 625 + k * 125, 125)])
        return 0
    lax.fori_loop(0, 5, dr, 0)


def _make_agg(nchunks):
    return functools.partial(
        pl.kernel,
        out_type=jax.ShapeDtypeStruct((2 * N_NODES, F_HALF), jnp.float32),
        mesh=plsc.VectorSubcoreMesh(**_MESH),
        scratch_types=[
            pltpu.VMEM((nchunks, 128), jnp.int32),
            pltpu.VMEM((nchunks, 128), jnp.int32),
            pltpu.VMEM((nchunks, 128), jnp.float32),
            pltpu.VMEM((128, F_HALF), jnp.float32),
            pltpu.VMEM((125, F_HALF), jnp.float32),
            pltpu.VMEM_SHARED((N_NODES, F_HALF), jnp.float32),
            pltpu.SemaphoreType.DMA,
        ],
    )(_agg_body)


def _mm1_body(x_ref, w_ref, da_ref, db_ref, o_ref):
    dinv = lax.rsqrt(da_ref[...] + db_ref[...] + 1.0)
    h = lax.dot_general(x_ref[...], w_ref[...], (((1,), (1,)), ((), ())),
                        precision=lax.Precision.HIGHEST,
                        preferred_element_type=jnp.float32)
    o_ref[...] = h * dinv


def _tc_first(x, W1, dega, degb):
    return pl.pallas_call(
        _mm1_body,
        grid=(2, NB),
        in_specs=[
            pl.BlockSpec((R_BLK, 128), lambda j, i: (i, 0)),
            pl.BlockSpec((128, 128), lambda j, i: (j, 0)),
            pl.BlockSpec((R_BLK, 1), lambda j, i: (i, 0)),
            pl.BlockSpec((R_BLK, 1), lambda j, i: (i, 0)),
        ],
        out_specs=pl.BlockSpec((R_BLK, F_HALF), lambda j, i: (j * NB + i, 0)),
        out_shape=jax.ShapeDtypeStruct((2 * N_NODES, F_HALF), jnp.float32),
    )(x, W1, dega, degb)


def _mid_body(a0, a1, g0, g1, da, db, b_ref, bw, bb, brm, brv, w2_ref, o_ref):
    dinv = lax.rsqrt(da[...] + db[...] + 1.0)
    s = bw[...] / jnp.sqrt(brv[...] + EPS)
    t = bb[...] - brm[...] * s
    b = b_ref[...]
    z0 = jnp.maximum((a0[...] + g0[...]) * dinv + b[:, :128], 0.0)
    z1 = jnp.maximum((a1[...] + g1[...]) * dinv + b[:, 128:], 0.0)
    z = jnp.concatenate([z0, z1], axis=1) * s + t
    h = lax.dot_general(z, w2_ref[...], (((1,), (1,)), ((), ())),
                        precision=lax.Precision.HIGHEST,
                        preferred_element_type=jnp.float32)
    o_ref[...] = h * dinv


def _tc_mid(acc, g, dega, degb, b1, bn_w, bn_b, bn_rm, bn_rv, W2):
    row = lambda j, i: (i, 0)
    hi = lambda j, i: (NB + i, 0)
    vec = lambda j, i: (0, 0)
    return pl.pallas_call(
        _mid_body,
        grid=(2, NB),
        in_specs=[
            pl.BlockSpec((R_BLK, F_HALF), row),
            pl.BlockSpec((R_BLK, F_HALF), hi),
            pl.BlockSpec((R_BLK, F_HALF), row),
            pl.BlockSpec((R_BLK, F_HALF), hi),
            pl.BlockSpec((R_BLK, 1), row),
            pl.BlockSpec((R_BLK, 1), row),
            pl.BlockSpec((1, 256), vec),
            pl.BlockSpec((1, 256), vec),
            pl.BlockSpec((1, 256), vec),
            pl.BlockSpec((1, 256), vec),
            pl.BlockSpec((1, 256), vec),
            pl.BlockSpec((128, 256), lambda j, i: (j, 0)),
        ],
        out_specs=pl.BlockSpec((R_BLK, F_HALF), lambda j, i: (j * NB + i, 0)),
        out_shape=jax.ShapeDtypeStruct((2 * N_NODES, F_HALF), jnp.float32),
    )(acc, acc, g, g, dega, degb, b1[None, :], bn_w[None, :], bn_b[None, :],
      bn_rm[None, :], bn_rv[None, :], W2)


def _fin_body(a0, a1, g0, g1, da, db, b_ref, bw, bb, brm, brv,
              lw_ref, lb_ref, o_ref):
    dinv = lax.rsqrt(da[...] + db[...] + 1.0)
    s = bw[...] / jnp.sqrt(brv[...] + EPS)
    t = bb[...] - brm[...] * s
    b = b_ref[...]
    z0 = jnp.maximum((a0[...] + g0[...]) * dinv + b[:, :128], 0.0)
    z1 = jnp.maximum((a1[...] + g1[...]) * dinv + b[:, 128:], 0.0)
    z = jnp.concatenate([z0, z1], axis=1) * s + t
    h = lax.dot_general(z, lw_ref[...], (((1,), (1,)), ((), ())),
                        precision=lax.Precision.HIGHEST,
                        preferred_element_type=jnp.float32)
    o_ref[...] = h + lb_ref[...]


def _tc_final(acc, g, dega, degb, b2, bn_w, bn_b, bn_rm, bn_rv, lin_w, lin_b):
    row = lambda i: (i, 0)
    hi = lambda i: (NB + i, 0)
    vec = lambda i: (0, 0)
    return pl.pallas_call(
        _fin_body,
        grid=(NB,),
        in_specs=[
            pl.BlockSpec((R_BLK, F_HALF), row),
            pl.BlockSpec((R_BLK, F_HALF), hi),
            pl.BlockSpec((R_BLK, F_HALF), row),
            pl.BlockSpec((R_BLK, F_HALF), hi),
            pl.BlockSpec((R_BLK, 1), row),
            pl.BlockSpec((R_BLK, 1), row),
            pl.BlockSpec((1, 256), vec),
            pl.BlockSpec((1, 256), vec),
            pl.BlockSpec((1, 256), vec),
            pl.BlockSpec((1, 256), vec),
            pl.BlockSpec((1, 256), vec),
            pl.BlockSpec((64, 256), vec),
            pl.BlockSpec((1, 64), vec),
        ],
        out_specs=pl.BlockSpec((R_BLK, 64), row),
        out_shape=jax.ShapeDtypeStruct((N_NODES, 64), jnp.float32),
    )(acc, acc, g, g, dega, degb, b2[None, :], bn_w[None, :], bn_b[None, :],
      bn_rm[None, :], bn_rv[None, :], lin_w, lin_b[None, :])


def _pad_edges(src, dst, ew, granule):
    e = src.shape[0]
    e_pad = ((e + granule - 1) // granule) * granule
    pad = e_pad - e
    if pad:
        fill = (jnp.arange(pad, dtype=jnp.int32) * 37) % N_NODES
        src = jnp.concatenate([src, fill])
        dst = jnp.concatenate([dst, fill])
        ew = jnp.concatenate([ew, jnp.zeros((pad,), ew.dtype)])
    return src, dst, ew, e_pad


def kernel(x, edge_index, edge_weight, W1, b1, W2, b2, lin_w, lin_b,
           bn1_w, bn1_b, bn1_rm, bn1_rv, bn2_w, bn2_b, bn2_rm, bn2_rv):
    src = edge_index[0].astype(jnp.int32)
    dst = edge_index[1].astype(jnp.int32)
    ew = edge_weight.astype(jnp.float32)

    sD, dD, wD, epD = _pad_edges(src, dst, ew, NC * NS * 128)
    cD = epD // (NC * NS * 128)
    dstD = dD.reshape(NC, NS, cD, 128)
    ewD = wD.reshape(NC, NS, cD, 128)

    sA, dA, wA, epA = _pad_edges(src, dst, ew, NS * 128)
    cA = epA // (NS * 128)
    sA3 = sA.reshape(NS, cA, 128)
    srco = jnp.stack([sA3, sA3 + N_NODES])
    dst3 = dA.reshape(NS, cA, 128)
    ew3 = wA.reshape(NS, cA, 128)

    degp = _make_deg(cD)(dstD, ewD)
    dega = degp[0, :N_NODES, None]
    degb = degp[1, :N_NODES, None]

    agg = _make_agg(cA)

    g1 = _tc_first(x, W1, dega, degb)
    acc1 = agg(g1, srco, dst3, ew3)
    g2 = _tc_mid(acc1, g1, dega, degb, b1, bn1_w, bn1_b, bn1_rm, bn1_rv, W2)
    acc2 = agg(g2, srco, dst3, ew3)
    return _tc_final(acc2, g2, dega, degb, b2, bn2_w, bn2_b, bn2_rm, bn2_rv,
                     lin_w, lin_b)


# speedup vs baseline: 4.9277x; 4.9277x over previous
import functools

import jax
import jax.numpy as jnp
from jax import lax
from jax.experimental import pallas as pl
from jax.experimental.pallas import tpu as pltpu
from jax.experimental.pallas import tpu_sc as plsc

N_NODES = 10000
N_PAD = 10240
F_QTR = 64
R_BLK = 1000
NB = N_NODES // R_BLK
EPS = 1e-5

_MESH = dict(core_axis_name="c", subcore_axis_name="s")
NC, NS = 2, 16


def _deg_body(dst_hbm, ew_hbm, out_hbm, dst_v, ew_v, zbuf, acc, sem):
    c = lax.axis_index("c")
    s = lax.axis_index("s")
    pltpu.sync_copy(dst_hbm.at[c, s], dst_v)
    pltpu.sync_copy(ew_hbm.at[c, s], ew_v)

    def zb(i, _):
        zbuf[pl.ds(i * 16, 16)] = jnp.zeros((16,), jnp.float32)
        return 0
    lax.fori_loop(0, 40, zb, 0)
    pltpu.sync_copy(zbuf, acc.at[pl.ds(s * 640, 640)])
    plsc.subcore_barrier()

    nchunks = dst_v.shape[0]

    def body(j, _):
        pltpu.sync_copy(ew_v.at[j], acc.at[dst_v.at[j]], add=True)
        return 0
    lax.fori_loop(0, nchunks, body, 0)
    plsc.subcore_barrier()
    pltpu.sync_copy(acc.at[pl.ds(s * 640, 640)],
                    out_hbm.at[pl.ds(c * N_PAD + s * 640, 640)])


def _make_deg(nchunks):
    return functools.partial(
        pl.kernel,
        out_type=jax.ShapeDtypeStruct((NC * N_PAD,), jnp.float32),
        mesh=plsc.VectorSubcoreMesh(**_MESH),
        compiler_params=pltpu.CompilerParams(use_tc_tiling_on_sc=False),
        scratch_types=[
            pltpu.VMEM((nchunks, 128), jnp.int32),
            pltpu.VMEM((nchunks, 128), jnp.float32),
            pltpu.VMEM((640,), jnp.float32),
            pltpu.VMEM_SHARED((N_PAD,), jnp.float32),
            pltpu.SemaphoreType.DMA,
        ],
    )(_deg_body)


def _agg_body(g_hbm, src_hbm, dst_hbm, ew_hbm, out_hbm,
              src_v, dst_v, ew_v, rows_v, zbuf, acc, sem):
    c = lax.axis_index("c")
    s = lax.axis_index("s")
    pltpu.sync_copy(dst_hbm.at[s], dst_v)
    pltpu.sync_copy(ew_hbm.at[s], ew_v)

    def zb(r, _):
        for f in range(4):
            zbuf[r, pl.ds(f * 16, 16)] = jnp.zeros((16,), jnp.float32)
        return 0
    lax.fori_loop(0, 128, zb, 0)

    nchunks = dst_v.shape[0]

    for p in range(2):
        q = 2 * c + p
        pltpu.sync_copy(src_hbm.at[q, s], src_v)
        for k in range(5):
            pltpu.sync_copy(zbuf, acc.at[pl.ds(s * 640 + k * 128, 128)])
        plsc.subcore_barrier()

        def chunk(j, _):
            pltpu.async_copy(g_hbm.at[src_v.at[j]], rows_v, sem).wait()

            def grp(gi, _2):
                wv = ew_v[j, pl.ds(gi * 16, 16)]
                e0 = gi * 16
                for l in range(16):
                    w = wv[l]
                    for f in range(4):
                        sl = pl.ds(f * 16, 16)
                        rows_v[e0 + l, sl] = rows_v[e0 + l, sl] * w
                return 0
            lax.fori_loop(0, 8, grp, 0)
            pltpu.sync_copy(rows_v, acc.at[dst_v.at[j]], add=True)
            return 0
        lax.fori_loop(0, nchunks, chunk, 0)
        plsc.subcore_barrier()

        def dr(k, _):
            pltpu.sync_copy(
                acc.at[pl.ds(s * 640 + k * 128, 128)],
                out_hbm.at[pl.ds(q * N_PAD + s * 640 + k * 128, 128)])
            return 0
        lax.fori_loop(0, 5, dr, 0)


def _make_agg(nchunks):
    return functools.partial(
        pl.kernel,
        out_type=jax.ShapeDtypeStruct((4 * N_PAD, F_QTR), jnp.float32),
        mesh=plsc.VectorSubcoreMesh(**_MESH),
        compiler_params=pltpu.CompilerParams(use_tc_tiling_on_sc=False),
        scratch_types=[
            pltpu.VMEM((nchunks, 128), jnp.int32),
            pltpu.VMEM((nchunks, 128), jnp.int32),
            pltpu.VMEM((nchunks, 128), jnp.float32),
            pltpu.VMEM((128, F_QTR), jnp.float32),
            pltpu.VMEM((128, F_QTR), jnp.float32),
            pltpu.VMEM_SHARED((N_PAD, F_QTR), jnp.float32),
            pltpu.SemaphoreType.DMA,
        ],
    )(_agg_body)


def _mm1_body(x_ref, w_ref, da_ref, db_ref, o_ref):
    dinv = lax.rsqrt(da_ref[...] + db_ref[...] + 1.0)
    h = lax.dot_general(x_ref[...], w_ref[...], (((1,), (1,)), ((), ())),
                        precision=lax.Precision.HIGHEST,
                        preferred_element_type=jnp.float32)
    o_ref[...] = h * dinv


def _tc_first(x, W1, dega, degb):
    return pl.pallas_call(
        _mm1_body,
        grid=(4, NB),
        in_specs=[
            pl.BlockSpec((R_BLK, 128), lambda j, i: (i, 0)),
            pl.BlockSpec((F_QTR, 128), lambda j, i: (j, 0)),
            pl.BlockSpec((R_BLK, 1), lambda j, i: (i, 0)),
            pl.BlockSpec((R_BLK, 1), lambda j, i: (i, 0)),
        ],
        out_specs=pl.BlockSpec((R_BLK, F_QTR), lambda j, i: (j * NB + i, 0)),
        out_shape=jax.ShapeDtypeStruct((4 * N_NODES, F_QTR), jnp.float32),
    )(x, W1, dega, degb)


def _zcat(aq, gq, dinv, b, s, t):
    zs = []
    for q in range(4):
        pre = (aq[q][...] + gq[q][...]) * dinv + b[:, q * 64:(q + 1) * 64]
        zs.append(jnp.maximum(pre, 0.0))
    return jnp.concatenate(zs, axis=1) * s + t


def _mid_body(a0, a1, a2, a3, g0, g1, g2, g3, da, db,
              b_ref, bw, bb, brm, brv, w2_ref, o_ref):
    dinv = lax.rsqrt(da[...] + db[...] + 1.0)
    s = bw[...] / jnp.sqrt(brv[...] + EPS)
    t = bb[...] - brm[...] * s
    z = _zcat((a0, a1, a2, a3), (g0, g1, g2, g3), dinv, b_ref[...], s, t)
    h = lax.dot_general(z, w2_ref[...], (((1,), (1,)), ((), ())),
                        precision=lax.Precision.HIGHEST,
                        preferred_element_type=jnp.float32)
    o_ref[...] = h * dinv


def _tc_mid(accq, g, dega, degb, b1, bn_w, bn_b, bn_rm, bn_rv, W2):
    row = lambda j, i: (i, 0)
    vec = lambda j, i: (0, 0)
    gq = lambda q: (lambda j, i, q=q: (q * NB + i, 0))
    return pl.pallas_call(
        _mid_body,
        grid=(4, NB),
        in_specs=[
            pl.BlockSpec((R_BLK, F_QTR), row),
            pl.BlockSpec((R_BLK, F_QTR), row),
            pl.BlockSpec((R_BLK, F_QTR), row),
            pl.BlockSpec((R_BLK, F_QTR), row),
            pl.BlockSpec((R_BLK, F_QTR), gq(0)),
            pl.BlockSpec((R_BLK, F_QTR), gq(1)),
            pl.BlockSpec((R_BLK, F_QTR), gq(2)),
            pl.BlockSpec((R_BLK, F_QTR), gq(3)),
            pl.BlockSpec((R_BLK, 1), row),
            pl.BlockSpec((R_BLK, 1), row),
            pl.BlockSpec((1, 256), vec),
            pl.BlockSpec((1, 256), vec),
            pl.BlockSpec((1, 256), vec),
            pl.BlockSpec((1, 256), vec),
            pl.BlockSpec((1, 256), vec),
            pl.BlockSpec((F_QTR, 256), lambda j, i: (j, 0)),
        ],
        out_specs=pl.BlockSpec((R_BLK, F_QTR), lambda j, i: (j * NB + i, 0)),
        out_shape=jax.ShapeDtypeStruct((4 * N_NODES, F_QTR), jnp.float32),
    )(*accq, g, g, g, g, dega, degb, b1[None, :], bn_w[None, :],
      bn_b[None, :], bn_rm[None, :], bn_rv[None, :], W2)


def _fin_body(a0, a1, a2, a3, g0, g1, g2, g3, da, db,
              b_ref, bw, bb, brm, brv, lw_ref, lb_ref, o_ref):
    dinv = lax.rsqrt(da[...] + db[...] + 1.0)
    s = bw[...] / jnp.sqrt(brv[...] + EPS)
    t = bb[...] - brm[...] * s
    z = _zcat((a0, a1, a2, a3), (g0, g1, g2, g3), dinv, b_ref[...], s, t)
    h = lax.dot_general(z, lw_ref[...], (((1,), (1,)), ((), ())),
                        precision=lax.Precision.HIGHEST,
                        preferred_element_type=jnp.float32)
    o_ref[...] = h + lb_ref[...]


def _tc_final(accq, g, dega, degb, b2, bn_w, bn_b, bn_rm, bn_rv,
              lin_w, lin_b):
    row = lambda i: (i, 0)
    vec = lambda i: (0, 0)
    gq = lambda q: (lambda i, q=q: (q * NB + i, 0))
    return pl.pallas_call(
        _fin_body,
        grid=(NB,),
        in_specs=[
            pl.BlockSpec((R_BLK, F_QTR), row),
            pl.BlockSpec((R_BLK, F_QTR), row),
            pl.BlockSpec((R_BLK, F_QTR), row),
            pl.BlockSpec((R_BLK, F_QTR), row),
            pl.BlockSpec((R_BLK, F_QTR), gq(0)),
            pl.BlockSpec((R_BLK, F_QTR), gq(1)),
            pl.BlockSpec((R_BLK, F_QTR), gq(2)),
            pl.BlockSpec((R_BLK, F_QTR), gq(3)),
            pl.BlockSpec((R_BLK, 1), row),
            pl.BlockSpec((R_BLK, 1), row),
            pl.BlockSpec((1, 256), vec),
            pl.BlockSpec((1, 256), vec),
            pl.BlockSpec((1, 256), vec),
            pl.BlockSpec((1, 256), vec),
            pl.BlockSpec((1, 256), vec),
            pl.BlockSpec((64, 256), vec),
            pl.BlockSpec((1, 64), vec),
        ],
        out_specs=pl.BlockSpec((R_BLK, 64), row),
        out_shape=jax.ShapeDtypeStruct((N_NODES, 64), jnp.float32),
    )(*accq, g, g, g, g, dega, degb, b2[None, :], bn_w[None, :],
      bn_b[None, :], bn_rm[None, :], bn_rv[None, :], lin_w, lin_b[None, :])


def _pad_edges(src, dst, ew, granule):
    e = src.shape[0]
    e_pad = ((e + granule - 1) // granule) * granule
    pad = e_pad - e
    if pad:
        fill = (jnp.arange(pad, dtype=jnp.int32) * 37) % N_NODES
        src = jnp.concatenate([src, fill])
        dst = jnp.concatenate([dst, fill])
        ew = jnp.concatenate([ew, jnp.zeros((pad,), ew.dtype)])
    return src, dst, ew, e_pad


def _quarters(accp):
    return tuple(accp[q * N_PAD:q * N_PAD + N_NODES] for q in range(4))


def kernel(x, edge_index, edge_weight, W1, b1, W2, b2, lin_w, lin_b,
           bn1_w, bn1_b, bn1_rm, bn1_rv, bn2_w, bn2_b, bn2_rm, bn2_rv):
    src = edge_index[0].astype(jnp.int32)
    dst = edge_index[1].astype(jnp.int32)
    ew = edge_weight.astype(jnp.float32)

    sD, dD, wD, epD = _pad_edges(src, dst, ew, NC * NS * 128)
    cD = epD // (NC * NS * 128)
    dstD = dD.reshape(NC, NS, cD, 128)
    ewD = wD.reshape(NC, NS, cD, 128)

    sA, dA, wA, epA = _pad_edges(src, dst, ew, NS * 128)
    cA = epA // (NS * 128)
    sA3 = sA.reshape(NS, cA, 128)
    srco = jnp.stack([sA3 + q * N_NODES for q in range(4)])
    dst3 = dA.reshape(NS, cA, 128)
    ew3 = wA.reshape(NS, cA, 128)

    degp = _make_deg(cD)(dstD, ewD)
    dega = degp[:N_NODES, None]
    degb = degp[N_PAD:N_PAD + N_NODES, None]

    agg = _make_agg(cA)

    g1 = _tc_first(x, W1, dega, degb)
    accp = agg(g1, srco, dst3, ew3)
    g2 = _tc_mid(_quarters(accp), g1, dega, degb, b1,
                 bn1_w, bn1_b, bn1_rm, bn1_rv, W2)
    accp2 = agg(g2, srco, dst3, ew3)
    return _tc_final(_quarters(accp2), g2, dega, degb, b2,
                     bn2_w, bn2_b, bn2_rm, bn2_rv, lin_w, lin_b)


# speedup vs baseline: 7.5611x; 1.5344x over previous
import functools

import jax
import jax.numpy as jnp
from jax import lax
from jax.experimental import pallas as pl
from jax.experimental.pallas import tpu as pltpu
from jax.experimental.pallas import tpu_sc as plsc

N_NODES = 10000
N_PAD = 10240
F_QTR = 64
R_BLK = 1000
NB = N_NODES // R_BLK
EPS = 1e-5

_MESH = dict(core_axis_name="c", subcore_axis_name="s")
NC, NS = 2, 16


def _deg_body(dst_hbm, ew_hbm, out_hbm, dst_v, ew_v, zbuf, acc, sem):
    c = lax.axis_index("c")
    s = lax.axis_index("s")
    pltpu.sync_copy(dst_hbm.at[c, s], dst_v)
    pltpu.sync_copy(ew_hbm.at[c, s], ew_v)

    def zb(i, _):
        zbuf[pl.ds(i * 16, 16)] = jnp.zeros((16,), jnp.float32)
        return 0
    lax.fori_loop(0, 40, zb, 0)
    pltpu.sync_copy(zbuf, acc.at[pl.ds(s * 640, 640)])
    plsc.subcore_barrier()

    nchunks = dst_v.shape[0]

    def body(j, _):
        pltpu.sync_copy(ew_v.at[j], acc.at[dst_v.at[j]], add=True)
        return 0
    lax.fori_loop(0, nchunks, body, 0)
    plsc.subcore_barrier()
    pltpu.sync_copy(acc.at[pl.ds(s * 640, 640)],
                    out_hbm.at[pl.ds(c * N_PAD + s * 640, 640)])


def _make_deg(nchunks):
    return functools.partial(
        pl.kernel,
        out_type=jax.ShapeDtypeStruct((NC * N_PAD,), jnp.float32),
        mesh=plsc.VectorSubcoreMesh(**_MESH),
        compiler_params=pltpu.CompilerParams(use_tc_tiling_on_sc=False),
        scratch_types=[
            pltpu.VMEM((nchunks, 128), jnp.int32),
            pltpu.VMEM((nchunks, 128), jnp.float32),
            pltpu.VMEM((640,), jnp.float32),
            pltpu.VMEM_SHARED((N_PAD,), jnp.float32),
            pltpu.SemaphoreType.DMA,
        ],
    )(_deg_body)


def _agg_body(g_hbm, src_hbm, dst_hbm, ew_hbm, out_hbm,
              src_v, dst_v, ew_v, gb0, gb1, acc, gs0, gs1):
    c = lax.axis_index("c")
    s = lax.axis_index("s")
    pltpu.sync_copy(dst_hbm.at[s], dst_v)
    pltpu.sync_copy(ew_hbm.at[s], ew_v)

    nchunks = dst_v.shape[0]
    gbufs = (gb0, gb1)
    gsems = (gs0, gs1)

    def scale(j, gb):
        def grp(gi, _2):
            wv = ew_v[j, pl.ds(gi * 16, 16)]
            e0 = gi * 16
            for l in range(16):
                w = wv[l]
                for f in range(4):
                    sl = pl.ds(f * 16, 16)
                    gb[e0 + l, sl] = gb[e0 + l, sl] * w
            return 0
        lax.fori_loop(0, 8, grp, 0)

    for p in range(2):
        q = 2 * c + p
        pltpu.sync_copy(src_hbm.at[q, s], src_v.at[pl.ds(0, nchunks)])
        for f in range(8):
            src_v[nchunks, pl.ds(f * 16, 16)] = jnp.zeros((16,), jnp.int32)
            src_v[nchunks + 1, pl.ds(f * 16, 16)] = jnp.zeros((16,), jnp.int32)

        def zb(r, _):
            for f in range(4):
                gb0[r, pl.ds(f * 16, 16)] = jnp.zeros((16,), jnp.float32)
            return 0
        lax.fori_loop(0, 128, zb, 0)
        for k in range(5):
            pltpu.sync_copy(gb0, acc.at[pl.ds(s * 640 + k * 128, 128)])
        plsc.subcore_barrier()

        pltpu.async_copy(g_hbm.at[src_v.at[0]], gb0, gs0)
        pltpu.async_copy(g_hbm.at[src_v.at[1]], gb1, gs1)

        def pair(k, _):
            for b in range(2):
                j = 2 * k + b
                gb = gbufs[b]
                pltpu.make_async_copy(g_hbm.at[src_v.at[j]], gb,
                                      gsems[b]).wait()
                scale(j, gb)
                pltpu.sync_copy(gb, acc.at[dst_v.at[j]], add=True)
                pltpu.async_copy(g_hbm.at[src_v.at[j + 2]], gb, gsems[b])
            return 0
        lax.fori_loop(0, nchunks // 2, pair, 0)

        for b in range(2):
            pltpu.make_async_copy(g_hbm.at[src_v.at[0]], gbufs[b],
                                  gsems[b]).wait()
        plsc.subcore_barrier()

        def dr(k, _):
            pltpu.sync_copy(
                acc.at[pl.ds(s * 640 + k * 128, 128)],
                out_hbm.at[pl.ds(q * N_PAD + s * 640 + k * 128, 128)])
            return 0
        lax.fori_loop(0, 5, dr, 0)


def _make_agg(nchunks):
    return functools.partial(
        pl.kernel,
        out_type=jax.ShapeDtypeStruct((4 * N_PAD, F_QTR), jnp.float32),
        mesh=plsc.VectorSubcoreMesh(**_MESH),
        compiler_params=pltpu.CompilerParams(use_tc_tiling_on_sc=False),
        scratch_types=[
            pltpu.VMEM((nchunks + 2, 128), jnp.int32),
            pltpu.VMEM((nchunks, 128), jnp.int32),
            pltpu.VMEM((nchunks, 128), jnp.float32),
            pltpu.VMEM((128, F_QTR), jnp.float32),
            pltpu.VMEM((128, F_QTR), jnp.float32),
            pltpu.VMEM_SHARED((N_PAD, F_QTR), jnp.float32),
            pltpu.SemaphoreType.DMA,
            pltpu.SemaphoreType.DMA,
        ],
    )(_agg_body)


def _mm1_body(x_ref, w_ref, da_ref, db_ref, o_ref):
    dinv = lax.rsqrt(da_ref[...] + db_ref[...] + 1.0)
    h = lax.dot_general(x_ref[...], w_ref[...], (((1,), (1,)), ((), ())),
                        precision=lax.Precision.HIGHEST,
                        preferred_element_type=jnp.float32)
    o_ref[...] = h * dinv


def _tc_first(x, W1, dega, degb):
    return pl.pallas_call(
        _mm1_body,
        grid=(4, NB),
        in_specs=[
            pl.BlockSpec((R_BLK, 128), lambda j, i: (i, 0)),
            pl.BlockSpec((F_QTR, 128), lambda j, i: (j, 0)),
            pl.BlockSpec((R_BLK, 1), lambda j, i: (i, 0)),
            pl.BlockSpec((R_BLK, 1), lambda j, i: (i, 0)),
        ],
        out_specs=pl.BlockSpec((R_BLK, F_QTR), lambda j, i: (j * NB + i, 0)),
        out_shape=jax.ShapeDtypeStruct((4 * N_NODES, F_QTR), jnp.float32),
    )(x, W1, dega, degb)


def _zcat(aq, gq, dinv, b, s, t):
    zs = []
    for q in range(4):
        pre = (aq[q][...] + gq[q][...]) * dinv + b[:, q * 64:(q + 1) * 64]
        zs.append(jnp.maximum(pre, 0.0))
    return jnp.concatenate(zs, axis=1) * s + t


def _mid_body(a0, a1, a2, a3, g0, g1, g2, g3, da, db,
              b_ref, bw, bb, brm, brv, w2_ref, o_ref):
    dinv = lax.rsqrt(da[...] + db[...] + 1.0)
    s = bw[...] / jnp.sqrt(brv[...] + EPS)
    t = bb[...] - brm[...] * s
    z = _zcat((a0, a1, a2, a3), (g0, g1, g2, g3), dinv, b_ref[...], s, t)
    h = lax.dot_general(z, w2_ref[...], (((1,), (1,)), ((), ())),
                        precision=lax.Precision.HIGHEST,
                        preferred_element_type=jnp.float32)
    o_ref[...] = h * dinv


def _tc_mid(accq, g, dega, degb, b1, bn_w, bn_b, bn_rm, bn_rv, W2):
    row = lambda j, i: (i, 0)
    vec = lambda j, i: (0, 0)
    gq = lambda q: (lambda j, i, q=q: (q * NB + i, 0))
    return pl.pallas_call(
        _mid_body,
        grid=(4, NB),
        in_specs=[
            pl.BlockSpec((R_BLK, F_QTR), row),
            pl.BlockSpec((R_BLK, F_QTR), row),
            pl.BlockSpec((R_BLK, F_QTR), row),
            pl.BlockSpec((R_BLK, F_QTR), row),
            pl.BlockSpec((R_BLK, F_QTR), gq(0)),
            pl.BlockSpec((R_BLK, F_QTR), gq(1)),
            pl.BlockSpec((R_BLK, F_QTR), gq(2)),
            pl.BlockSpec((R_BLK, F_QTR), gq(3)),
            pl.BlockSpec((R_BLK, 1), row),
            pl.BlockSpec((R_BLK, 1), row),
            pl.BlockSpec((1, 256), vec),
            pl.BlockSpec((1, 256), vec),
            pl.BlockSpec((1, 256), vec),
            pl.BlockSpec((1, 256), vec),
            pl.BlockSpec((1, 256), vec),
            pl.BlockSpec((F_QTR, 256), lambda j, i: (j, 0)),
        ],
        out_specs=pl.BlockSpec((R_BLK, F_QTR), lambda j, i: (j * NB + i, 0)),
        out_shape=jax.ShapeDtypeStruct((4 * N_NODES, F_QTR), jnp.float32),
    )(*accq, g, g, g, g, dega, degb, b1[None, :], bn_w[None, :],
      bn_b[None, :], bn_rm[None, :], bn_rv[None, :], W2)


def _fin_body(a0, a1, a2, a3, g0, g1, g2, g3, da, db,
              b_ref, bw, bb, brm, brv, lw_ref, lb_ref, o_ref):
    dinv = lax.rsqrt(da[...] + db[...] + 1.0)
    s = bw[...] / jnp.sqrt(brv[...] + EPS)
    t = bb[...] - brm[...] * s
    z = _zcat((a0, a1, a2, a3), (g0, g1, g2, g3), dinv, b_ref[...], s, t)
    h = lax.dot_general(z, lw_ref[...], (((1,), (1,)), ((), ())),
                        precision=lax.Precision.HIGHEST,
                        preferred_element_type=jnp.float32)
    o_ref[...] = h + lb_ref[...]


def _tc_final(accq, g, dega, degb, b2, bn_w, bn_b, bn_rm, bn_rv,
              lin_w, lin_b):
    row = lambda i: (i, 0)
    vec = lambda i: (0, 0)
    gq = lambda q: (lambda i, q=q: (q * NB + i, 0))
    return pl.pallas_call(
        _fin_body,
        grid=(NB,),
        in_specs=[
            pl.BlockSpec((R_BLK, F_QTR), row),
            pl.BlockSpec((R_BLK, F_QTR), row),
            pl.BlockSpec((R_BLK, F_QTR), row),
            pl.BlockSpec((R_BLK, F_QTR), row),
            pl.BlockSpec((R_BLK, F_QTR), gq(0)),
            pl.BlockSpec((R_BLK, F_QTR), gq(1)),
            pl.BlockSpec((R_BLK, F_QTR), gq(2)),
            pl.BlockSpec((R_BLK, F_QTR), gq(3)),
            pl.BlockSpec((R_BLK, 1), row),
            pl.BlockSpec((R_BLK, 1), row),
            pl.BlockSpec((1, 256), vec),
            pl.BlockSpec((1, 256), vec),
            pl.BlockSpec((1, 256), vec),
            pl.BlockSpec((1, 256), vec),
            pl.BlockSpec((1, 256), vec),
            pl.BlockSpec((64, 256), vec),
            pl.BlockSpec((1, 64), vec),
        ],
        out_specs=pl.BlockSpec((R_BLK, 64), row),
        out_shape=jax.ShapeDtypeStruct((N_NODES, 64), jnp.float32),
    )(*accq, g, g, g, g, dega, degb, b2[None, :], bn_w[None, :],
      bn_b[None, :], bn_rm[None, :], bn_rv[None, :], lin_w, lin_b[None, :])


def _pad_edges(src, dst, ew, granule):
    e = src.shape[0]
    e_pad = ((e + granule - 1) // granule) * granule
    pad = e_pad - e
    if pad:
        fill = (jnp.arange(pad, dtype=jnp.int32) * 37) % N_NODES
        src = jnp.concatenate([src, fill])
        dst = jnp.concatenate([dst, fill])
        ew = jnp.concatenate([ew, jnp.zeros((pad,), ew.dtype)])
    return src, dst, ew, e_pad


def _quarters(accp):
    return tuple(accp[q * N_PAD:q * N_PAD + N_NODES] for q in range(4))


def kernel(x, edge_index, edge_weight, W1, b1, W2, b2, lin_w, lin_b,
           bn1_w, bn1_b, bn1_rm, bn1_rv, bn2_w, bn2_b, bn2_rm, bn2_rv):
    src = edge_index[0].astype(jnp.int32)
    dst = edge_index[1].astype(jnp.int32)
    ew = edge_weight.astype(jnp.float32)

    sD, dD, wD, epD = _pad_edges(src, dst, ew, NC * NS * 128)
    cD = epD // (NC * NS * 128)
    dstD = dD.reshape(NC, NS, cD, 128)
    ewD = wD.reshape(NC, NS, cD, 128)

    sA, dA, wA, epA = _pad_edges(src, dst, ew, NS * 256)
    cA = epA // (NS * 128)
    sA3 = sA.reshape(NS, cA, 128)
    srco = jnp.stack([sA3 + q * N_NODES for q in range(4)])
    dst3 = dA.reshape(NS, cA, 128)
    ew3 = wA.reshape(NS, cA, 128)

    degp = _make_deg(cD)(dstD, ewD)
    dega = degp[:N_NODES, None]
    degb = degp[N_PAD:N_PAD + N_NODES, None]

    agg = _make_agg(cA)

    g1 = _tc_first(x, W1, dega, degb)
    accp = agg(g1, srco, dst3, ew3)
    g2 = _tc_mid(_quarters(accp), g1, dega, degb, b1,
                 bn1_w, bn1_b, bn1_rm, bn1_rv, W2)
    accp2 = agg(g2, srco, dst3, ew3)
    return _tc_final(_quarters(accp2), g2, dega, degb, b2,
                     bn2_w, bn2_b, bn2_rm, bn2_rv, lin_w, lin_b)


# speedup vs baseline: 8.7080x; 1.1517x over previous
import functools

import jax
import jax.numpy as jnp
from jax import lax
from jax.experimental import pallas as pl
from jax.experimental.pallas import tpu as pltpu
from jax.experimental.pallas import tpu_sc as plsc

N_NODES = 10000
N_PAD = 10240
F_QTR = 64
R_BLK = 1000
CHK = 112
NB = N_NODES // R_BLK
EPS = 1e-5

_MESH = dict(core_axis_name="c", subcore_axis_name="s")
NC, NS = 2, 16


def _deg_body(dst_hbm, ew_hbm, out_hbm, db0, db1, eb0, eb1, zbuf, acc,
              ds0, ds1, es0, es1):
    c = lax.axis_index("c")
    s = lax.axis_index("s")

    def zb(i, _):
        zbuf[pl.ds(i * 16, 16)] = jnp.zeros((16,), jnp.float32)
        return 0
    lax.fori_loop(0, 40, zb, 0)
    pltpu.sync_copy(zbuf, acc.at[pl.ds(s * 640, 640)])
    plsc.subcore_barrier()

    nchunks = dst_hbm.shape[2]
    dbufs, ebufs = (db0, db1), (eb0, eb1)
    dsems, esems = (ds0, ds1), (es0, es1)

    pltpu.async_copy(dst_hbm.at[c, s, 0], db0, ds0)
    pltpu.async_copy(ew_hbm.at[c, s, 0], eb0, es0)
    pltpu.async_copy(dst_hbm.at[c, s, 1], db1, ds1)
    pltpu.async_copy(ew_hbm.at[c, s, 1], eb1, es1)

    def pair(k, _):
        for b in range(2):
            j = 2 * k + b
            pltpu.make_async_copy(dst_hbm.at[c, s, 0], dbufs[b],
                                  dsems[b]).wait()
            pltpu.make_async_copy(ew_hbm.at[c, s, 0], ebufs[b],
                                  esems[b]).wait()
            pltpu.sync_copy(ebufs[b], acc.at[dbufs[b]], add=True)
            nxt = jnp.minimum(j + 2, nchunks - 1)
            pltpu.async_copy(dst_hbm.at[c, s, nxt], dbufs[b], dsems[b])
            pltpu.async_copy(ew_hbm.at[c, s, nxt], ebufs[b], esems[b])
        return 0
    lax.fori_loop(0, nchunks // 2, pair, 0)
    for b in range(2):
        pltpu.make_async_copy(dst_hbm.at[c, s, 0], dbufs[b], dsems[b]).wait()
        pltpu.make_async_copy(ew_hbm.at[c, s, 0], ebufs[b], esems[b]).wait()
    plsc.subcore_barrier()
    pltpu.sync_copy(acc.at[pl.ds(s * 640, 640)],
                    out_hbm.at[pl.ds(c * N_PAD + s * 640, 640)])


def _make_deg(nchunks):
    return functools.partial(
        pl.kernel,
        out_type=jax.ShapeDtypeStruct((NC * N_PAD,), jnp.float32),
        mesh=plsc.VectorSubcoreMesh(**_MESH),
        compiler_params=pltpu.CompilerParams(use_tc_tiling_on_sc=False),
        scratch_types=[
            pltpu.VMEM((128,), jnp.int32),
            pltpu.VMEM((128,), jnp.int32),
            pltpu.VMEM((128,), jnp.float32),
            pltpu.VMEM((128,), jnp.float32),
            pltpu.VMEM((640,), jnp.float32),
            pltpu.VMEM_SHARED((N_PAD,), jnp.float32),
            pltpu.SemaphoreType.DMA,
            pltpu.SemaphoreType.DMA,
            pltpu.SemaphoreType.DMA,
            pltpu.SemaphoreType.DMA,
        ],
    )(_deg_body)


def _agg_body(g_hbm, src_hbm, dst_hbm, ew_hbm, out_hbm,
              src_v, dst_v, ew_v, gb0, gb1, sb0, sb1, acc,
              gs0, gs1, ss0, ss1):
    c = lax.axis_index("c")
    s = lax.axis_index("s")
    pltpu.sync_copy(dst_hbm.at[s], dst_v)
    pltpu.sync_copy(ew_hbm.at[s], ew_v)

    nchunks = dst_v.shape[0]
    gbufs, sbufs = (gb0, gb1), (sb0, sb1)
    gsems, ssems = (gs0, gs1), (ss0, ss1)

    def scale(j, gb, sb):
        def grp(gi, _2):
            wv = ew_v[j, pl.ds(gi * 16, 16)]
            e0 = gi * 16
            for l in range(16):
                w = wv[l]
                for f in range(4):
                    sl = pl.ds(f * 16, 16)
                    sb[e0 + l, sl] = gb[e0 + l, sl] * w
            return 0
        lax.fori_loop(0, CHK // 16, grp, 0)

    for p in range(2):
        q = 2 * c + p
        pltpu.sync_copy(src_hbm.at[q, s], src_v.at[pl.ds(0, nchunks)])
        for f in range(CHK // 16):
            src_v[nchunks, pl.ds(f * 16, 16)] = jnp.zeros((16,), jnp.int32)
            src_v[nchunks + 1, pl.ds(f * 16, 16)] = jnp.zeros((16,), jnp.int32)

        def zb(r, _):
            for f in range(4):
                gb0[r, pl.ds(f * 16, 16)] = jnp.zeros((16,), jnp.float32)
            return 0
        lax.fori_loop(0, CHK, zb, 0)
        for k in range(10):
            pltpu.sync_copy(gb0.at[pl.ds(0, 64)],
                            acc.at[pl.ds(s * 640 + k * 64, 64)])
        plsc.subcore_barrier()

        cp0 = pltpu.async_copy(g_hbm.at[src_v.at[0]], gb0, gs0)
        cp1 = pltpu.async_copy(g_hbm.at[src_v.at[1]], gb1, gs1)
        cp0.wait()
        scale(0, gb0, sb0)
        pltpu.async_copy(g_hbm.at[src_v.at[2]], gb0, gs0)
        pltpu.async_copy(sb0, acc.at[dst_v.at[0]], ss0, add=True)
        cp1.wait()
        scale(1, gb1, sb1)
        pltpu.async_copy(g_hbm.at[src_v.at[3]], gb1, gs1)
        pltpu.async_copy(sb1, acc.at[dst_v.at[1]], ss1, add=True)

        def pair(k, _):
            for b in range(2):
                j = 2 * k + b
                gb, sb = gbufs[b], sbufs[b]
                pltpu.make_async_copy(g_hbm.at[src_v.at[j]], gb,
                                      gsems[b]).wait()
                pltpu.make_async_copy(sb, acc.at[dst_v.at[j]],
                                      ssems[b]).wait()
                scale(j, gb, sb)
                pltpu.async_copy(g_hbm.at[src_v.at[j + 2]], gb, gsems[b])
                pltpu.async_copy(sb, acc.at[dst_v.at[j]], ssems[b], add=True)
            return 0
        lax.fori_loop(1, nchunks // 2, pair, 0)

        for b in range(2):
            pltpu.make_async_copy(g_hbm.at[src_v.at[0]], gbufs[b],
                                  gsems[b]).wait()
            pltpu.make_async_copy(sbufs[b], acc.at[dst_v.at[0]],
                                  ssems[b]).wait()
        plsc.subcore_barrier()

        def dr(k, _):
            pltpu.sync_copy(
                acc.at[pl.ds(s * 640 + k * 64, 64)],
                out_hbm.at[pl.ds(q * N_PAD + s * 640 + k * 64, 64)])
            return 0
        lax.fori_loop(0, 10, dr, 0)


def _make_agg(nchunks):
    return functools.partial(
        pl.kernel,
        out_type=jax.ShapeDtypeStruct((4 * N_PAD, F_QTR), jnp.float32),
        mesh=plsc.VectorSubcoreMesh(**_MESH),
        compiler_params=pltpu.CompilerParams(use_tc_tiling_on_sc=False),
        scratch_types=[
            pltpu.VMEM((nchunks + 2, CHK), jnp.int32),
            pltpu.VMEM((nchunks, CHK), jnp.int32),
            pltpu.VMEM((nchunks, CHK), jnp.float32),
            pltpu.VMEM((CHK, F_QTR), jnp.float32),
            pltpu.VMEM((CHK, F_QTR), jnp.float32),
            pltpu.VMEM((CHK, F_QTR), jnp.float32),
            pltpu.VMEM((CHK, F_QTR), jnp.float32),
            pltpu.VMEM_SHARED((N_PAD, F_QTR), jnp.float32),
            pltpu.SemaphoreType.DMA,
            pltpu.SemaphoreType.DMA,
            pltpu.SemaphoreType.DMA,
            pltpu.SemaphoreType.DMA,
        ],
    )(_agg_body)


def _mm1_body(x_ref, w_ref, da_ref, db_ref, o_ref):
    dinv = lax.rsqrt(da_ref[...] + db_ref[...] + 1.0)
    h = lax.dot_general(x_ref[...], w_ref[...], (((1,), (1,)), ((), ())),
                        precision=lax.Precision.HIGHEST,
                        preferred_element_type=jnp.float32)
    o_ref[...] = h * dinv


def _tc_first(x, W1, dega, degb):
    return pl.pallas_call(
        _mm1_body,
        grid=(4, NB),
        in_specs=[
            pl.BlockSpec((R_BLK, 128), lambda j, i: (i, 0)),
            pl.BlockSpec((F_QTR, 128), lambda j, i: (j, 0)),
            pl.BlockSpec((R_BLK, 1), lambda j, i: (i, 0)),
            pl.BlockSpec((R_BLK, 1), lambda j, i: (i, 0)),
        ],
        out_specs=pl.BlockSpec((R_BLK, F_QTR), lambda j, i: (j * NB + i, 0)),
        out_shape=jax.ShapeDtypeStruct((4 * N_NODES, F_QTR), jnp.float32),
    )(x, W1, dega, degb)


def _zcat(aq, gq, dinv, b, s, t):
    zs = []
    for q in range(4):
        pre = (aq[q][...] + gq[q][...]) * dinv + b[:, q * 64:(q + 1) * 64]
        zs.append(jnp.maximum(pre, 0.0))
    return jnp.concatenate(zs, axis=1) * s + t


def _mid_body(a0, a1, a2, a3, g0, g1, g2, g3, da, db,
              b_ref, bw, bb, brm, brv, w2_ref, o_ref):
    dinv = lax.rsqrt(da[...] + db[...] + 1.0)
    s = bw[...] / jnp.sqrt(brv[...] + EPS)
    t = bb[...] - brm[...] * s
    z = _zcat((a0, a1, a2, a3), (g0, g1, g2, g3), dinv, b_ref[...], s, t)
    h = lax.dot_general(z, w2_ref[...], (((1,), (1,)), ((), ())),
                        precision=lax.Precision.HIGHEST,
                        preferred_element_type=jnp.float32)
    o_ref[...] = h * dinv


def _tc_mid(accq, g, dega, degb, b1, bn_w, bn_b, bn_rm, bn_rv, W2):
    row = lambda j, i: (i, 0)
    vec = lambda j, i: (0, 0)
    gq = lambda q: (lambda j, i, q=q: (q * NB + i, 0))
    return pl.pallas_call(
        _mid_body,
        grid=(4, NB),
        in_specs=[
            pl.BlockSpec((R_BLK, F_QTR), row),
            pl.BlockSpec((R_BLK, F_QTR), row),
            pl.BlockSpec((R_BLK, F_QTR), row),
            pl.BlockSpec((R_BLK, F_QTR), row),
            pl.BlockSpec((R_BLK, F_QTR), gq(0)),
            pl.BlockSpec((R_BLK, F_QTR), gq(1)),
            pl.BlockSpec((R_BLK, F_QTR), gq(2)),
            pl.BlockSpec((R_BLK, F_QTR), gq(3)),
            pl.BlockSpec((R_BLK, 1), row),
            pl.BlockSpec((R_BLK, 1), row),
            pl.BlockSpec((1, 256), vec),
            pl.BlockSpec((1, 256), vec),
            pl.BlockSpec((1, 256), vec),
            pl.BlockSpec((1, 256), vec),
            pl.BlockSpec((1, 256), vec),
            pl.BlockSpec((F_QTR, 256), lambda j, i: (j, 0)),
        ],
        out_specs=pl.BlockSpec((R_BLK, F_QTR), lambda j, i: (j * NB + i, 0)),
        out_shape=jax.ShapeDtypeStruct((4 * N_NODES, F_QTR), jnp.float32),
    )(*accq, g, g, g, g, dega, degb, b1[None, :], bn_w[None, :],
      bn_b[None, :], bn_rm[None, :], bn_rv[None, :], W2)


def _fin_body(a0, a1, a2, a3, g0, g1, g2, g3, da, db,
              b_ref, bw, bb, brm, brv, lw_ref, lb_ref, o_ref):
    dinv = lax.rsqrt(da[...] + db[...] + 1.0)
    s = bw[...] / jnp.sqrt(brv[...] + EPS)
    t = bb[...] - brm[...] * s
    z = _zcat((a0, a1, a2, a3), (g0, g1, g2, g3), dinv, b_ref[...], s, t)
    h = lax.dot_general(z, lw_ref[...], (((1,), (1,)), ((), ())),
                        precision=lax.Precision.HIGHEST,
                        preferred_element_type=jnp.float32)
    o_ref[...] = h + lb_ref[...]


def _tc_final(accq, g, dega, degb, b2, bn_w, bn_b, bn_rm, bn_rv,
              lin_w, lin_b):
    row = lambda i: (i, 0)
    vec = lambda i: (0, 0)
    gq = lambda q: (lambda i, q=q: (q * NB + i, 0))
    return pl.pallas_call(
        _fin_body,
        grid=(NB,),
        in_specs=[
            pl.BlockSpec((R_BLK, F_QTR), row),
            pl.BlockSpec((R_BLK, F_QTR), row),
            pl.BlockSpec((R_BLK, F_QTR), row),
            pl.BlockSpec((R_BLK, F_QTR), row),
            pl.BlockSpec((R_BLK, F_QTR), gq(0)),
            pl.BlockSpec((R_BLK, F_QTR), gq(1)),
            pl.BlockSpec((R_BLK, F_QTR), gq(2)),
            pl.BlockSpec((R_BLK, F_QTR), gq(3)),
            pl.BlockSpec((R_BLK, 1), row),
            pl.BlockSpec((R_BLK, 1), row),
            pl.BlockSpec((1, 256), vec),
            pl.BlockSpec((1, 256), vec),
            pl.BlockSpec((1, 256), vec),
            pl.BlockSpec((1, 256), vec),
            pl.BlockSpec((1, 256), vec),
            pl.BlockSpec((64, 256), vec),
            pl.BlockSpec((1, 64), vec),
        ],
        out_specs=pl.BlockSpec((R_BLK, 64), row),
        out_shape=jax.ShapeDtypeStruct((N_NODES, 64), jnp.float32),
    )(*accq, g, g, g, g, dega, degb, b2[None, :], bn_w[None, :],
      bn_b[None, :], bn_rm[None, :], bn_rv[None, :], lin_w, lin_b[None, :])


def _pad_edges(src, dst, ew, granule):
    e = src.shape[0]
    e_pad = ((e + granule - 1) // granule) * granule
    pad = e_pad - e
    if pad:
        fill = (jnp.arange(pad, dtype=jnp.int32) * 37) % N_NODES
        src = jnp.concatenate([src, fill])
        dst = jnp.concatenate([dst, fill])
        ew = jnp.concatenate([ew, jnp.zeros((pad,), ew.dtype)])
    return src, dst, ew, e_pad


def _quarters(accp):
    return tuple(accp[q * N_PAD:q * N_PAD + N_NODES] for q in range(4))


def kernel(x, edge_index, edge_weight, W1, b1, W2, b2, lin_w, lin_b,
           bn1_w, bn1_b, bn1_rm, bn1_rv, bn2_w, bn2_b, bn2_rm, bn2_rv):
    src = edge_index[0].astype(jnp.int32)
    dst = edge_index[1].astype(jnp.int32)
    ew = edge_weight.astype(jnp.float32)

    sD, dD, wD, epD = _pad_edges(src, dst, ew, NC * NS * 256)
    cD = epD // (NC * NS * 128)
    dstD = dD.reshape(NC, NS, cD, 128)
    ewD = wD.reshape(NC, NS, cD, 128)

    sA, dA, wA, epA = _pad_edges(src, dst, ew, NS * CHK * 2)
    cA = epA // (NS * CHK)
    sA3 = sA.reshape(NS, cA, CHK)
    srco = jnp.stack([sA3 + q * N_NODES for q in range(4)])
    dst3 = dA.reshape(NS, cA, CHK)
    ew3 = wA.reshape(NS, cA, CHK)

    degp = _make_deg(cD)(dstD, ewD)
    dega = degp[:N_NODES, None]
    degb = degp[N_PAD:N_PAD + N_NODES, None]

    agg = _make_agg(cA)

    g1 = _tc_first(x, W1, dega, degb)
    accp = agg(g1, srco, dst3, ew3)
    g2 = _tc_mid(_quarters(accp), g1, dega, degb, b1,
                 bn1_w, bn1_b, bn1_rm, bn1_rv, W2)
    accp2 = agg(g2, srco, dst3, ew3)
    return _tc_final(_quarters(accp2), g2, dega, degb, b2,
                     bn2_w, bn2_b, bn2_rm, bn2_rv, lin_w, lin_b)


# speedup vs baseline: 11.3113x; 1.2990x over previous
import functools

import jax
import jax.numpy as jnp
from jax import lax
from jax.experimental import pallas as pl
from jax.experimental.pallas import tpu as pltpu
from jax.experimental.pallas import tpu_sc as plsc

N_NODES = 10000
N_PAD = 10240
F_QTR = 64
R_BLK = 1000
CHK = 128
WIN = 8
NB = N_NODES // R_BLK
EPS = 1e-5

_MESH = dict(core_axis_name="c", subcore_axis_name="s")
NC, NS = 2, 16


def _deg_body(dst_hbm, ew_hbm, out_hbm, db0, db1, eb0, eb1, zbuf, acc,
              ds0, ds1, es0, es1):
    c = lax.axis_index("c")
    s = lax.axis_index("s")

    def zb(i, _):
        zbuf[pl.ds(i * 16, 16)] = jnp.zeros((16,), jnp.float32)
        return 0
    lax.fori_loop(0, 40, zb, 0)
    pltpu.sync_copy(zbuf, acc.at[pl.ds(s * 640, 640)])
    plsc.subcore_barrier()

    nchunks = dst_hbm.shape[2]
    dbufs, ebufs = (db0, db1), (eb0, eb1)
    dsems, esems = (ds0, ds1), (es0, es1)

    pltpu.async_copy(dst_hbm.at[c, s, 0], db0, ds0)
    pltpu.async_copy(ew_hbm.at[c, s, 0], eb0, es0)
    pltpu.async_copy(dst_hbm.at[c, s, 1], db1, ds1)
    pltpu.async_copy(ew_hbm.at[c, s, 1], eb1, es1)

    def pair(k, _):
        for b in range(2):
            j = 2 * k + b
            pltpu.make_async_copy(dst_hbm.at[c, s, 0], dbufs[b],
                                  dsems[b]).wait()
            pltpu.make_async_copy(ew_hbm.at[c, s, 0], ebufs[b],
                                  esems[b]).wait()
            pltpu.sync_copy(ebufs[b], acc.at[dbufs[b]], add=True)
            nxt = jnp.minimum(j + 2, nchunks - 1)
            pltpu.async_copy(dst_hbm.at[c, s, nxt], dbufs[b], dsems[b])
            pltpu.async_copy(ew_hbm.at[c, s, nxt], ebufs[b], esems[b])
        return 0
    lax.fori_loop(0, nchunks // 2, pair, 0)
    for b in range(2):
        pltpu.make_async_copy(dst_hbm.at[c, s, 0], dbufs[b], dsems[b]).wait()
        pltpu.make_async_copy(ew_hbm.at[c, s, 0], ebufs[b], esems[b]).wait()
    plsc.subcore_barrier()
    pltpu.sync_copy(acc.at[pl.ds(s * 640, 640)],
                    out_hbm.at[pl.ds(c * N_PAD + s * 640, 640)])


def _make_deg(nchunks):
    return functools.partial(
        pl.kernel,
        out_type=jax.ShapeDtypeStruct((NC * N_PAD,), jnp.float32),
        mesh=plsc.VectorSubcoreMesh(**_MESH),
        compiler_params=pltpu.CompilerParams(use_tc_tiling_on_sc=False),
        scratch_types=[
            pltpu.VMEM((128,), jnp.int32),
            pltpu.VMEM((128,), jnp.int32),
            pltpu.VMEM((128,), jnp.float32),
            pltpu.VMEM((128,), jnp.float32),
            pltpu.VMEM((640,), jnp.float32),
            pltpu.VMEM_SHARED((N_PAD,), jnp.float32),
            pltpu.SemaphoreType.DMA,
            pltpu.SemaphoreType.DMA,
            pltpu.SemaphoreType.DMA,
            pltpu.SemaphoreType.DMA,
        ],
    )(_deg_body)


def _agg_body(g_hbm, src_hbm, dst_hbm, ew_hbm, out_hbm,
              sw0, sw1, dw0, dw1, eww0, eww1, gb0, gb1, sb0, sb1, tbl, acc,
              gs0, gs1, ss0, ss1, ws0, ws1):
    c = lax.axis_index("c")
    s = lax.axis_index("s")
    npieces = src_hbm.shape[1] // WIN
    sws, dws, ews_ = (sw0, sw1), (dw0, dw1), (eww0, eww1)
    gbufs, sbufs = (gb0, gb1), (sb0, sb1)
    gsems, ssems = (gs0, gs1), (ss0, ss1)
    wsems = (ws0, ws1)

    def load_win(piece, h):
        sl = pl.ds(piece * WIN, WIN)
        pltpu.async_copy(src_hbm.at[s, sl], sws[h], wsems[h])
        pltpu.async_copy(dst_hbm.at[s, sl], dws[h], wsems[h])
        pltpu.async_copy(ew_hbm.at[s, sl], ews_[h], wsems[h])

    def wait_win(h):
        sl = pl.ds(0, WIN)
        pltpu.make_async_copy(src_hbm.at[s, sl], sws[h], wsems[h]).wait()
        pltpu.make_async_copy(dst_hbm.at[s, sl], dws[h], wsems[h]).wait()
        pltpu.make_async_copy(ew_hbm.at[s, sl], ews_[h], wsems[h]).wait()

    def scale(ewb, t, gb, sb):
        def grp(gi, _2):
            wv = ewb[t, pl.ds(gi * 16, 16)]
            e0 = gi * 16
            for l in range(16):
                w = wv[l]
                for f in range(4):
                    slf = pl.ds(f * 16, 16)
                    sb[e0 + l, slf] = gb[e0 + l, slf] * w
            return 0
        lax.fori_loop(0, CHK // 16, grp, 0)

    t0 = jnp.where(s < NS - 1, s * 640, N_NODES - 640)

    for p in range(2):
        q = 2 * c + p
        pltpu.sync_copy(g_hbm.at[pl.ds(q * N_NODES + t0, 640)],
                        tbl.at[pl.ds(t0, 640)])

        def zb(r, _):
            for f in range(4):
                gb0[r, pl.ds(f * 16, 16)] = jnp.zeros((16,), jnp.float32)
            return 0
        lax.fori_loop(0, CHK, zb, 0)
        for k in range(10):
            pltpu.sync_copy(gb0.at[pl.ds(0, 64)],
                            acc.at[pl.ds(s * 640 + k * 64, 64)])

        load_win(0, 0)
        load_win(1, 1)
        wait_win(0)
        plsc.subcore_barrier()
        pltpu.async_copy(tbl.at[sw0.at[0]], gb0, gs0)
        pltpu.async_copy(tbl.at[sw0.at[1]], gb1, gs1)

        def piece_pair(u, _):
            for h in range(2):
                sw, dw, ewb = sws[h], dws[h], ews_[h]
                swn = sws[1 - h]
                for t in range(WIN):
                    b = t % 2
                    gb, sb = gbufs[b], sbufs[b]
                    pltpu.make_async_copy(tbl.at[sw.at[0]], gb,
                                          gsems[b]).wait()
                    if t >= 2:
                        pltpu.make_async_copy(sb, acc.at[dw.at[0]],
                                              ssems[b]).wait()
                    scale(ewb, t, gb, sb)
                    if t < WIN - 2:
                        pltpu.async_copy(tbl.at[sw.at[t + 2]], gb, gsems[b])
                    else:
                        pltpu.async_copy(tbl.at[swn.at[t - (WIN - 2)]], gb,
                                         gsems[b])
                    pltpu.async_copy(sb, acc.at[dw.at[t]], ssems[b],
                                     add=True)
                    if t == WIN - 3:
                        wait_win(1 - h)
                for b in range(2):
                    pltpu.make_async_copy(sbufs[b], acc.at[dw.at[0]],
                                          ssems[b]).wait()
                nxt = jnp.minimum(2 * u + h + 2, npieces - 1)
                load_win(nxt, h)
            return 0
        lax.fori_loop(0, npieces // 2, piece_pair, 0)

        for b in range(2):
            pltpu.make_async_copy(tbl.at[sw0.at[0]], gbufs[b],
                                  gsems[b]).wait()
        wait_win(1)
        plsc.subcore_barrier()

        def dr(k, _):
            pltpu.sync_copy(
                acc.at[pl.ds(s * 640 + k * 64, 64)],
                out_hbm.at[pl.ds(q * N_PAD + s * 640 + k * 64, 64)])
            return 0
        lax.fori_loop(0, 10, dr, 0)


def _make_agg(nchunks):
    return functools.partial(
        pl.kernel,
        out_type=jax.ShapeDtypeStruct((4 * N_PAD, F_QTR), jnp.float32),
        mesh=plsc.VectorSubcoreMesh(**_MESH),
        compiler_params=pltpu.CompilerParams(use_tc_tiling_on_sc=False),
        scratch_types=[
            pltpu.VMEM((WIN, CHK), jnp.int32),
            pltpu.VMEM((WIN, CHK), jnp.int32),
            pltpu.VMEM((WIN, CHK), jnp.int32),
            pltpu.VMEM((WIN, CHK), jnp.int32),
            pltpu.VMEM((WIN, CHK), jnp.float32),
            pltpu.VMEM((WIN, CHK), jnp.float32),
            pltpu.VMEM((CHK, F_QTR), jnp.float32),
            pltpu.VMEM((CHK, F_QTR), jnp.float32),
            pltpu.VMEM((CHK, F_QTR), jnp.float32),
            pltpu.VMEM((CHK, F_QTR), jnp.float32),
            pltpu.VMEM_SHARED((N_NODES, F_QTR), jnp.float32),
            pltpu.VMEM_SHARED((N_PAD, F_QTR), jnp.float32),
            pltpu.SemaphoreType.DMA,
            pltpu.SemaphoreType.DMA,
            pltpu.SemaphoreType.DMA,
            pltpu.SemaphoreType.DMA,
            pltpu.SemaphoreType.DMA,
            pltpu.SemaphoreType.DMA,
        ],
    )(_agg_body)


def _mm1_body(x_ref, w_ref, da_ref, db_ref, o_ref):
    dinv = lax.rsqrt(da_ref[...] + db_ref[...] + 1.0)
    h = lax.dot_general(x_ref[...], w_ref[...], (((1,), (1,)), ((), ())),
                        precision=lax.Precision.HIGHEST,
                        preferred_element_type=jnp.float32)
    o_ref[...] = h * dinv


def _tc_first(x, W1, dega, degb):
    return pl.pallas_call(
        _mm1_body,
        grid=(4, NB),
        in_specs=[
            pl.BlockSpec((R_BLK, 128), lambda j, i: (i, 0)),
            pl.BlockSpec((F_QTR, 128), lambda j, i: (j, 0)),
            pl.BlockSpec((R_BLK, 1), lambda j, i: (i, 0)),
            pl.BlockSpec((R_BLK, 1), lambda j, i: (i, 0)),
        ],
        out_specs=pl.BlockSpec((R_BLK, F_QTR), lambda j, i: (j * NB + i, 0)),
        out_shape=jax.ShapeDtypeStruct((4 * N_NODES, F_QTR), jnp.float32),
    )(x, W1, dega, degb)


def _zcat(aq, gq, dinv, b, s, t):
    zs = []
    for q in range(4):
        pre = (aq[q][...] + gq[q][...]) * dinv + b[:, q * 64:(q + 1) * 64]
        zs.append(jnp.maximum(pre, 0.0))
    return jnp.concatenate(zs, axis=1) * s + t


def _mid_body(a0, a1, a2, a3, g0, g1, g2, g3, da, db,
              b_ref, bw, bb, brm, brv, w2_ref, o_ref):
    dinv = lax.rsqrt(da[...] + db[...] + 1.0)
    s = bw[...] / jnp.sqrt(brv[...] + EPS)
    t = bb[...] - brm[...] * s
    z = _zcat((a0, a1, a2, a3), (g0, g1, g2, g3), dinv, b_ref[...], s, t)
    h = lax.dot_general(z, w2_ref[...], (((1,), (1,)), ((), ())),
                        precision=lax.Precision.HIGHEST,
                        preferred_element_type=jnp.float32)
    o_ref[...] = h * dinv


def _tc_mid(accq, g, dega, degb, b1, bn_w, bn_b, bn_rm, bn_rv, W2):
    row = lambda j, i: (i, 0)
    vec = lambda j, i: (0, 0)
    gq = lambda q: (lambda j, i, q=q: (q * NB + i, 0))
    return pl.pallas_call(
        _mid_body,
        grid=(4, NB),
        in_specs=[
            pl.BlockSpec((R_BLK, F_QTR), row),
            pl.BlockSpec((R_BLK, F_QTR), row),
            pl.BlockSpec((R_BLK, F_QTR), row),
            pl.BlockSpec((R_BLK, F_QTR), row),
            pl.BlockSpec((R_BLK, F_QTR), gq(0)),
            pl.BlockSpec((R_BLK, F_QTR), gq(1)),
            pl.BlockSpec((R_BLK, F_QTR), gq(2)),
            pl.BlockSpec((R_BLK, F_QTR), gq(3)),
            pl.BlockSpec((R_BLK, 1), row),
            pl.BlockSpec((R_BLK, 1), row),
            pl.BlockSpec((1, 256), vec),
            pl.BlockSpec((1, 256), vec),
            pl.BlockSpec((1, 256), vec),
            pl.BlockSpec((1, 256), vec),
            pl.BlockSpec((1, 256), vec),
            pl.BlockSpec((F_QTR, 256), lambda j, i: (j, 0)),
        ],
        out_specs=pl.BlockSpec((R_BLK, F_QTR), lambda j, i: (j * NB + i, 0)),
        out_shape=jax.ShapeDtypeStruct((4 * N_NODES, F_QTR), jnp.float32),
    )(*accq, g, g, g, g, dega, degb, b1[None, :], bn_w[None, :],
      bn_b[None, :], bn_rm[None, :], bn_rv[None, :], W2)


def _last_body(g0, g1, g2, g3, da, db, lw_ref, lb_ref, o_ref):
    rsq = jnp.sqrt(da[...] + db[...] + 1.0)
    z = jnp.concatenate([g0[...], g1[...], g2[...], g3[...]], axis=1) * rsq
    h = lax.dot_general(z, lw_ref[...], (((1,), (1,)), ((), ())),
                        precision=lax.Precision.HIGHEST,
                        preferred_element_type=jnp.float32)
    o_ref[...] = h + lb_ref[...]


def _tc_last(g, dega, degb, lin_w, lin_b):
    row = lambda i: (i, 0)
    vec = lambda i: (0, 0)
    gq = lambda q: (lambda i, q=q: (q * NB + i, 0))
    return pl.pallas_call(
        _last_body,
        grid=(NB,),
        in_specs=[
            pl.BlockSpec((R_BLK, F_QTR), gq(0)),
            pl.BlockSpec((R_BLK, F_QTR), gq(1)),
            pl.BlockSpec((R_BLK, F_QTR), gq(2)),
            pl.BlockSpec((R_BLK, F_QTR), gq(3)),
            pl.BlockSpec((R_BLK, 1), row),
            pl.BlockSpec((R_BLK, 1), row),
            pl.BlockSpec((64, 256), vec),
            pl.BlockSpec((1, 64), vec),
        ],
        out_specs=pl.BlockSpec((R_BLK, 64), row),
        out_shape=jax.ShapeDtypeStruct((N_NODES, 64), jnp.float32),
    )(g, g, g, g, dega, degb, lin_w, lin_b[None, :])


def _pad_edges(src, dst, ew, granule):
    e = src.shape[0]
    e_pad = ((e + granule - 1) // granule) * granule
    pad = e_pad - e
    if pad:
        fill = (jnp.arange(pad, dtype=jnp.int32) * 37) % N_NODES
        src = jnp.concatenate([src, fill])
        dst = jnp.concatenate([dst, fill])
        ew = jnp.concatenate([ew, jnp.zeros((pad,), ew.dtype)])
    return src, dst, ew, e_pad


def _quarters(accp):
    return tuple(accp[q * N_PAD:q * N_PAD + N_NODES] for q in range(4))


def kernel(x, edge_index, edge_weight, W1, b1, W2, b2, lin_w, lin_b,
           bn1_w, bn1_b, bn1_rm, bn1_rv, bn2_w, bn2_b, bn2_rm, bn2_rv):
    src = edge_index[0].astype(jnp.int32)
    dst = edge_index[1].astype(jnp.int32)
    ew = edge_weight.astype(jnp.float32)

    sD, dD, wD, epD = _pad_edges(src, dst, ew, NC * NS * 256)
    cD = epD // (NC * NS * 128)
    dstD = dD.reshape(NC, NS, cD, 128)
    ewD = wD.reshape(NC, NS, cD, 128)

    sA, dA, wA, epA = _pad_edges(src, dst, ew, NS * CHK * 2 * WIN)
    cA = epA // (NS * CHK)
    src3 = sA.reshape(NS, cA, CHK)
    dst3 = dA.reshape(NS, cA, CHK)
    ew3 = wA.reshape(NS, cA, CHK)

    degp = _make_deg(cD)(dstD, ewD)
    dega = degp[:N_NODES, None]
    degb = degp[N_PAD:N_PAD + N_NODES, None]

    agg = _make_agg(cA)

    g1 = _tc_first(x, W1, dega, degb)

    eye = jnp.eye(W2.shape[0], dtype=jnp.float32)
    xs = (jnp.stack([W2, eye]), jnp.stack([b1, b2]),
          jnp.stack([bn1_w, bn2_w]), jnp.stack([bn1_b, bn2_b]),
          jnp.stack([bn1_rm, bn2_rm]), jnp.stack([bn1_rv, bn2_rv]))

    def body(g, x_l):
        Wl, bl, bwl, bbl, brml, brvl = x_l
        accp = agg(g, src3, dst3, ew3)
        g_next = _tc_mid(_quarters(accp), g, dega, degb, bl,
                         bwl, bbl, brml, brvl, Wl)
        return g_next, None

    gz, _ = lax.scan(body, g1, xs)
    return _tc_last(gz, dega, degb, lin_w, lin_b)
